# baseline (speedup 1.0000x reference)
import math

import jax
import jax.numpy as jnp
import numpy as np
from jax import lax
from jax.experimental import pallas as pl
from jax.experimental.pallas import tpu as pltpu
from jax.experimental.pallas import tpu_sc as plsc

_VOCAB = 1000000
_SIZE = 64
_MAX_SEQ_LEN = 1000
_BATCH = 4096
_SEQ = 200
_SCALE = np.float32(math.sqrt(_SIZE))

_NC = 2
_NS = 16
_NW = _NC * _NS

_BB = _BATCH // _NW
_NROWS = _BB * _SEQ
_CHUNKS = _BB


def _make_pe(max_seq_len, size):
    pe = np.zeros((max_seq_len, size), dtype=np.float32)
    position = np.arange(0, max_seq_len, dtype=np.float32)[:, None]
    div_term = np.exp(
        np.arange(0, size, 2, dtype=np.float32) * -(math.log(10000.0) / size))
    pe[:, 0::2] = np.sin(position * div_term)
    pe[:, 1::2] = np.cos(position * div_term)
    return pe


_PE = _make_pe(_MAX_SEQ_LEN, _SIZE)[:_SEQ]


def _idx_body(xt_hbm, out_hbm, xt_v, idxo_v):
    wid = lax.axis_index("s") * _NC + lax.axis_index("c")
    b0 = pl.multiple_of(wid * _BB, 128)
    pltpu.sync_copy(xt_hbm.at[:, pl.ds(b0, _BB)], xt_v)

    lanes = lax.iota(jnp.int32, 16)

    @pl.loop(0, _NROWS // 16)
    def _tr(m):
        d = m * 16 + lanes
        b_loc = d // _SEQ
        l = d - b_loc * _SEQ
        vals = plsc.load_gather(xt_v, [l, b_loc])
        idxo_v[m // 8, pl.ds((m % 8) * 16, 16)] = vals

    pltpu.sync_copy(idxo_v, out_hbm.at[pl.ds(wid * _SEQ, _SEQ)])


def _gather_body(table_hbm, idx_hbm, pe_hbm, out_hbm,
                 idx_v, gbuf0, gbuf1, obuf0, obuf1, pe_v,
                 gsem0, gsem1, ssem):
    wid = lax.axis_index("s") * _NC + lax.axis_index("c")
    row0 = wid * _NROWS
    gbuf = (gbuf0, gbuf1)
    obuf = (obuf0, obuf1)
    gsem = (gsem0, gsem1)

    pltpu.sync_copy(idx_hbm.at[pl.ds(pl.multiple_of(row0, 8), _NROWS)], idx_v)
    pltpu.sync_copy(pe_hbm, pe_v)

    def fire_gather(c, par):
        pltpu.async_copy(
            table_hbm.at[idx_v.at[pl.ds(c * _SEQ, _SEQ)]],
            gbuf[par], gsem[par])

    def wait_gather(par):
        pltpu.make_async_copy(
            table_hbm.at[pl.ds(0, _SEQ)], gbuf[par], gsem[par]).wait()

    def store(c, par):
        pltpu.async_copy(
            obuf[par], out_hbm.at[pl.ds(row0 + c * _SEQ, _SEQ)], ssem)

    def wait_store(par):
        pltpu.make_async_copy(
            obuf[par], out_hbm.at[pl.ds(0, _SEQ)], ssem).wait()

    def compute(par):
        gb, ob = gbuf[par], obuf[par]

        @pl.loop(0, _SEQ)
        def _pos(l):
            for k in range(_SIZE // 16):
                sl = pl.ds(k * 16, 16)
                ob[l, sl] = gb[l, sl] * _SCALE + pe_v[l, sl]

    fire_gather(0, 0)

    @pl.loop(0, _CHUNKS, step=2)
    def _outer(t):
        @pl.when(t > 0)
        def _():
            wait_store(1)
        fire_gather(t + 1, 1)
        wait_gather(0)
        compute(0)
        store(t, 0)

        wait_store(0)

        @pl.when(t + 2 < _CHUNKS)
        def _():
            fire_gather(t + 2, 0)
        wait_gather(1)
        compute(1)
        store(t + 1, 1)

    wait_store(1)


def kernel(x, emb_table):
    b, seq = x.shape
    assert (b, seq) == (_BATCH, _SEQ) and emb_table.shape == (_VOCAB, _SIZE)
    xt = x.astype(jnp.int32).T
    pe = jnp.asarray(_PE)

    idx_fmt = pl.kernel(
        _idx_body,
        out_type=jax.ShapeDtypeStruct((_NW * _SEQ, _BB), jnp.int32),
        mesh=plsc.VectorSubcoreMesh(core_axis_name="c", subcore_axis_name="s"),
        compiler_params=pltpu.CompilerParams(needs_layout_passes=False),
        scratch_types=[
            pltpu.VMEM((_SEQ, _BB), jnp.int32),
            pltpu.VMEM((_SEQ, _BB), jnp.int32),
        ],
    )
    idx2d = idx_fmt(xt)

    run = pl.kernel(
        _gather_body,
        out_type=jax.ShapeDtypeStruct((b * seq, 2 * _SIZE), jnp.float32),
        mesh=plsc.VectorSubcoreMesh(core_axis_name="c", subcore_axis_name="s"),
        compiler_params=pltpu.CompilerParams(
            use_tc_tiling_on_sc=False, needs_layout_passes=False),
        scratch_types=[
            pltpu.VMEM((_NROWS,), jnp.int32),
            pltpu.VMEM((_SEQ, _SIZE), jnp.float32),
            pltpu.VMEM((_SEQ, _SIZE), jnp.float32),
            pltpu.VMEM((_SEQ, 2 * _SIZE), jnp.float32),
            pltpu.VMEM((_SEQ, 2 * _SIZE), jnp.float32),
            pltpu.VMEM((_SEQ, _SIZE), jnp.float32),
            pltpu.SemaphoreType.DMA,
            pltpu.SemaphoreType.DMA,
            pltpu.SemaphoreType.DMA,
        ],
    )
    h128 = run(emb_table, idx2d.reshape(b * seq), pe)
    return h128[:, :_SIZE].reshape(b, seq, _SIZE)


# speedup vs baseline: 1.0025x; 1.0025x over previous
import math

import jax
import jax.numpy as jnp
import numpy as np
from jax import lax
from jax.experimental import pallas as pl
from jax.experimental.pallas import tpu as pltpu
from jax.experimental.pallas import tpu_sc as plsc

_VOCAB = 1000000
_SIZE = 64
_MAX_SEQ_LEN = 1000
_BATCH = 4096
_SEQ = 200
_SCALE = np.float32(math.sqrt(_SIZE))

_NC = 2
_NS = 16
_NW = _NC * _NS

_BB = _BATCH // _NW
_NROWS = _BB * _SEQ
_SPC = 1
_CHUNKS = _BB // _SPC
_GSPLIT = (104, 96)


def _make_pe(max_seq_len, size):
    pe = np.zeros((max_seq_len, size), dtype=np.float32)
    position = np.arange(0, max_seq_len, dtype=np.float32)[:, None]
    div_term = np.exp(
        np.arange(0, size, 2, dtype=np.float32) * -(math.log(10000.0) / size))
    pe[:, 0::2] = np.sin(position * div_term)
    pe[:, 1::2] = np.cos(position * div_term)
    return pe


_PE = _make_pe(_MAX_SEQ_LEN, _SIZE)[:_SEQ]


def _idx_body(xt_hbm, out_hbm, xt_v, idxo_v):
    wid = lax.axis_index("s") * _NC + lax.axis_index("c")
    b0 = pl.multiple_of(wid * _BB, 128)
    pltpu.sync_copy(xt_hbm.at[:, pl.ds(b0, _BB)], xt_v)

    lanes = lax.iota(jnp.int32, 16)

    @pl.loop(0, _NROWS // 16)
    def _tr(m):
        d = m * 16 + lanes
        b_loc = d // _SEQ
        l = d - b_loc * _SEQ
        vals = plsc.load_gather(xt_v, [l, b_loc])
        idxo_v[m // 8, pl.ds((m % 8) * 16, 16)] = vals

    pltpu.sync_copy(idxo_v, out_hbm.at[pl.ds(wid * _SEQ, _SEQ)])


def _gather_body(table_hbm, idx_hbm, pe_hbm, out_hbm,
                 idx_v, gbuf0, gbuf1, obuf0, obuf1, pe_v,
                 gsem0, gsem1, ssem):
    wid = lax.axis_index("s") * _NC + lax.axis_index("c")
    row0 = wid * _NROWS
    gbuf = (gbuf0, gbuf1)
    obuf = (obuf0, obuf1)
    gsem = (gsem0, gsem1)

    pltpu.sync_copy(idx_hbm.at[pl.ds(pl.multiple_of(row0, 8), _NROWS)], idx_v)
    pltpu.sync_copy(pe_hbm, pe_v)

    def fire_gather(c, par):
        off = 0
        for n in _GSPLIT:
            pltpu.async_copy(
                table_hbm.at[idx_v.at[pl.ds(c * _SEQ + off, n)]],
                gbuf[par].at[pl.ds(off, n)], gsem[par])
            off += n

    def wait_gather(par):
        pltpu.make_async_copy(
            table_hbm.at[pl.ds(0, _SPC * _SEQ)], gbuf[par], gsem[par]).wait()

    def store(c, par):
        pltpu.async_copy(
            obuf[par],
            out_hbm.at[pl.ds(row0 + c * _SPC * _SEQ, _SPC * _SEQ)], ssem)

    def wait_store(par):
        pltpu.make_async_copy(
            obuf[par], out_hbm.at[pl.ds(0, _SPC * _SEQ)], ssem).wait()

    def compute(par):
        gb, ob = gbuf[par], obuf[par]

        @pl.loop(0, _SEQ)
        def _pos(l):
            pes = [pe_v[l, pl.ds(k * 16, 16)] for k in range(_SIZE // 16)]
            for s in range(_SPC):
                r = s * _SEQ + l
                for k in range(_SIZE // 16):
                    sl = pl.ds(k * 16, 16)
                    ob[r, sl] = gb[r, sl] * _SCALE + pes[k]

    fire_gather(0, 0)

    @pl.loop(0, _CHUNKS, step=2)
    def _outer(t):
        @pl.when(t > 0)
        def _():
            wait_store(1)
        fire_gather(t + 1, 1)
        wait_gather(0)
        compute(0)
        store(t, 0)

        wait_store(0)

        @pl.when(t + 2 < _CHUNKS)
        def _():
            fire_gather(t + 2, 0)
        wait_gather(1)
        compute(1)
        store(t + 1, 1)

    wait_store(1)


def kernel(x, emb_table):
    b, seq = x.shape
    assert (b, seq) == (_BATCH, _SEQ) and emb_table.shape == (_VOCAB, _SIZE)
    xt = x.astype(jnp.int32).T
    pe = jnp.asarray(_PE)

    idx_fmt = pl.kernel(
        _idx_body,
        out_type=jax.ShapeDtypeStruct((_NW * _SEQ, _BB), jnp.int32),
        mesh=plsc.VectorSubcoreMesh(core_axis_name="c", subcore_axis_name="s"),
        compiler_params=pltpu.CompilerParams(needs_layout_passes=False),
        scratch_types=[
            pltpu.VMEM((_SEQ, _BB), jnp.int32),
            pltpu.VMEM((_SEQ, _BB), jnp.int32),
        ],
    )
    idx2d = idx_fmt(xt)

    run = pl.kernel(
        _gather_body,
        out_type=jax.ShapeDtypeStruct((b * seq, 2 * _SIZE), jnp.float32),
        mesh=plsc.VectorSubcoreMesh(core_axis_name="c", subcore_axis_name="s"),
        compiler_params=pltpu.CompilerParams(
            use_tc_tiling_on_sc=False, needs_layout_passes=False),
        scratch_types=[
            pltpu.VMEM((_NROWS,), jnp.int32),
            pltpu.VMEM((_SPC * _SEQ, _SIZE), jnp.float32),
            pltpu.VMEM((_SPC * _SEQ, _SIZE), jnp.float32),
            pltpu.VMEM((_SPC * _SEQ, 2 * _SIZE), jnp.float32),
            pltpu.VMEM((_SPC * _SEQ, 2 * _SIZE), jnp.float32),
            pltpu.VMEM((_SEQ, _SIZE), jnp.float32),
            pltpu.SemaphoreType.DMA,
            pltpu.SemaphoreType.DMA,
            pltpu.SemaphoreType.DMA,
        ],
    )
    h128 = run(emb_table, idx2d.reshape(b * seq), pe)
    return h128[:, :_SIZE].reshape(b, seq, _SIZE)


# speedup vs baseline: 1.0027x; 1.0002x over previous
import math

import jax
import jax.numpy as jnp
import numpy as np
from jax import lax
from jax.experimental import pallas as pl
from jax.experimental.pallas import tpu as pltpu
from jax.experimental.pallas import tpu_sc as plsc

_VOCAB = 1000000
_SIZE = 64
_MAX_SEQ_LEN = 1000
_BATCH = 4096
_SEQ = 200
_SCALE = np.float32(math.sqrt(_SIZE))

_NC = 2
_NS = 16
_NW = _NC * _NS

_BB = _BATCH // _NW
_NROWS = _BB * _SEQ
_SPC = 1
_CHUNKS = _BB // _SPC
_GSPLIT = (104, 96)


def _make_pe(max_seq_len, size):
    pe = np.zeros((max_seq_len, size), dtype=np.float32)
    position = np.arange(0, max_seq_len, dtype=np.float32)[:, None]
    div_term = np.exp(
        np.arange(0, size, 2, dtype=np.float32) * -(math.log(10000.0) / size))
    pe[:, 0::2] = np.sin(position * div_term)
    pe[:, 1::2] = np.cos(position * div_term)
    return pe


_PE = _make_pe(_MAX_SEQ_LEN, _SIZE)[:_SEQ]


def _idx_body(xt_hbm, out_hbm, xt_v, idxo_v):
    wid = lax.axis_index("s") * _NC + lax.axis_index("c")
    b0 = pl.multiple_of(wid * _BB, 128)
    pltpu.sync_copy(xt_hbm.at[:, pl.ds(b0, _BB)], xt_v)

    lanes = lax.iota(jnp.int32, 16)

    @pl.loop(0, _NROWS // 16)
    def _tr(m):
        d = m * 16 + lanes
        b_loc = d // _SEQ
        l = d - b_loc * _SEQ
        vals = plsc.load_gather(xt_v, [l, b_loc])
        idxo_v[m // 8, pl.ds((m % 8) * 16, 16)] = vals

    pltpu.sync_copy(idxo_v, out_hbm.at[pl.ds(wid * _SEQ, _SEQ)])


def _gather_body(table_hbm, idx_hbm, pe_hbm, out_hbm,
                 idx_v, gbuf0, gbuf1, obuf0, obuf1, pe_v,
                 gsem0, gsem1, ssem):
    wid = lax.axis_index("s") * _NC + lax.axis_index("c")
    row0 = wid * _NROWS
    gbuf = (gbuf0, gbuf1)
    obuf = (obuf0, obuf1)
    gsem = (gsem0, gsem1)

    pltpu.sync_copy(
        idx_hbm.at[pl.ds(pl.multiple_of(wid * _BB, 8), _BB)], idx_v)
    pltpu.sync_copy(pe_hbm, pe_v)

    def fire_gather(c, par):
        pltpu.async_copy(
            table_hbm.at[idx_v.at[c]], gbuf[par], gsem[par])

    def wait_gather(par):
        pltpu.make_async_copy(
            table_hbm.at[pl.ds(0, _SPC * _SEQ)], gbuf[par], gsem[par]).wait()

    def store(c, par):
        pltpu.async_copy(
            obuf[par],
            out_hbm.at[pl.ds(row0 + c * _SPC * _SEQ, _SPC * _SEQ)], ssem)

    def wait_store(par):
        pltpu.make_async_copy(
            obuf[par], out_hbm.at[pl.ds(0, _SPC * _SEQ)], ssem).wait()

    def compute(par):
        gb, ob = gbuf[par], obuf[par]

        @pl.loop(0, _SEQ)
        def _pos(l):
            pes = [pe_v[l, pl.ds(k * 16, 16)] for k in range(_SIZE // 16)]
            for s in range(_SPC):
                r = s * _SEQ + l
                for k in range(_SIZE // 16):
                    sl = pl.ds(k * 16, 16)
                    ob[r, sl] = gb[r, sl] * _SCALE + pes[k]

    fire_gather(0, 0)

    @pl.loop(0, _CHUNKS, step=2)
    def _outer(t):
        @pl.when(t > 0)
        def _():
            wait_store(1)
        fire_gather(t + 1, 1)
        wait_gather(0)
        compute(0)
        store(t, 0)

        wait_store(0)

        @pl.when(t + 2 < _CHUNKS)
        def _():
            fire_gather(t + 2, 0)
        wait_gather(1)
        compute(1)
        store(t + 1, 1)

    wait_store(1)


def kernel(x, emb_table):
    b, seq = x.shape
    assert (b, seq) == (_BATCH, _SEQ) and emb_table.shape == (_VOCAB, _SIZE)
    xt = x.astype(jnp.int32).T
    pe = jnp.asarray(_PE)

    idx_fmt = pl.kernel(
        _idx_body,
        out_type=jax.ShapeDtypeStruct((_NW * _SEQ, _BB), jnp.int32),
        mesh=plsc.VectorSubcoreMesh(core_axis_name="c", subcore_axis_name="s"),
        compiler_params=pltpu.CompilerParams(needs_layout_passes=False),
        scratch_types=[
            pltpu.VMEM((_SEQ, _BB), jnp.int32),
            pltpu.VMEM((_SEQ, _BB), jnp.int32),
        ],
    )
    idx2d = idx_fmt(xt)

    run = pl.kernel(
        _gather_body,
        out_type=jax.ShapeDtypeStruct((b * seq, 2 * _SIZE), jnp.float32),
        mesh=plsc.VectorSubcoreMesh(core_axis_name="c", subcore_axis_name="s"),
        compiler_params=pltpu.CompilerParams(
            use_tc_tiling_on_sc=False, needs_layout_passes=False),
        scratch_types=[
            pltpu.VMEM((_BB, _SEQ), jnp.int32),
            pltpu.VMEM((_SPC * _SEQ, _SIZE), jnp.float32),
            pltpu.VMEM((_SPC * _SEQ, _SIZE), jnp.float32),
            pltpu.VMEM((_SPC * _SEQ, 2 * _SIZE), jnp.float32),
            pltpu.VMEM((_SPC * _SEQ, 2 * _SIZE), jnp.float32),
            pltpu.VMEM((_SEQ, _SIZE), jnp.float32),
            pltpu.SemaphoreType.DMA,
            pltpu.SemaphoreType.DMA,
            pltpu.SemaphoreType.DMA,
        ],
    )
    h128 = run(emb_table, idx2d.reshape(b, seq), pe)
    return h128[:, :_SIZE].reshape(b, seq, _SIZE)


# speedup vs baseline: 1.2027x; 1.1995x over previous
import math

import jax
import jax.numpy as jnp
import numpy as np
from jax import lax
from jax.experimental import pallas as pl
from jax.experimental.pallas import tpu as pltpu
from jax.experimental.pallas import tpu_sc as plsc

_VOCAB = 1000000
_SIZE = 64
_MAX_SEQ_LEN = 1000
_BATCH = 4096
_SEQ = 200
_SCALE = np.float32(math.sqrt(_SIZE))

_NC = 2
_NS = 16
_NW = _NC * _NS

_SEQ_PER_W = _BATCH // _NW
_SEQ_PER_CHUNK = 2
_CHUNKS = _SEQ_PER_W // _SEQ_PER_CHUNK
_ROWS_PER_CHUNK = _SEQ_PER_CHUNK * _SEQ
_GATHER = 100
_NGATHER = _ROWS_PER_CHUNK // _GATHER
_IDX_ROWS = _SEQ_PER_W * _SEQ // _GATHER


def _make_pe(max_seq_len, size):
    pe = np.zeros((max_seq_len, size), dtype=np.float32)
    position = np.arange(0, max_seq_len, dtype=np.float32)[:, None]
    div_term = np.exp(
        np.arange(0, size, 2, dtype=np.float32) * -(math.log(10000.0) / size))
    pe[:, 0::2] = np.sin(position * div_term)
    pe[:, 1::2] = np.cos(position * div_term)
    return pe


_PE = _make_pe(_MAX_SEQ_LEN, _SIZE)[:_SEQ]


def _body(table_hbm, x_hbm, pe_hbm, out_hbm,
          idx_v, rows0, rows1, pe_v, gsem0, gsem1, ssem):
    wid = lax.axis_index("s") * _NC + lax.axis_index("c")
    rows = (rows0, rows1)
    gsem = (gsem0, gsem1)

    i0 = pl.multiple_of(wid * _IDX_ROWS, 8)
    pltpu.sync_copy(x_hbm.at[pl.ds(i0, _IDX_ROWS)], idx_v)
    pltpu.sync_copy(pe_hbm, pe_v)

    def fire_gathers(c, par):
        for g in range(_NGATHER):
            pltpu.async_copy(
                table_hbm.at[idx_v.at[c * _NGATHER + g]],
                rows[par].at[pl.ds(g * _GATHER, _GATHER)], gsem[par])

    def wait_gathers(par):
        pltpu.make_async_copy(
            table_hbm.at[pl.ds(0, _ROWS_PER_CHUNK)], rows[par],
            gsem[par]).wait()

    def store(c, par):
        row0 = pl.multiple_of((wid * _CHUNKS + c) * _ROWS_PER_CHUNK, 8)
        pltpu.async_copy(rows[par], out_hbm.at[pl.ds(row0, _ROWS_PER_CHUNK)],
                         ssem)

    def wait_store(par):
        pltpu.make_async_copy(
            rows[par], out_hbm.at[pl.ds(0, _ROWS_PER_CHUNK)], ssem).wait()

    def compute(par):
        buf = rows[par]

        @pl.loop(0, _SEQ)
        def _pos(l):
            pes = [pe_v[l, pl.ds(k * 16, 16)] for k in range(_SIZE // 16)]
            for s in range(_SEQ_PER_CHUNK):
                r = s * _SEQ + l
                for k in range(_SIZE // 16):
                    sl = pl.ds(k * 16, 16)
                    buf[r, sl] = buf[r, sl] * _SCALE + pes[k]

    fire_gathers(0, 0)

    @pl.loop(0, _CHUNKS, step=2)
    def _outer(t):
        @pl.when(t > 0)
        def _():
            wait_store(1)
        fire_gathers(t + 1, 1)
        wait_gathers(0)
        compute(0)
        store(t, 0)

        wait_store(0)

        @pl.when(t + 2 < _CHUNKS)
        def _():
            fire_gathers(t + 2, 0)
        wait_gathers(1)
        compute(1)
        store(t + 1, 1)

    wait_store(1)


def kernel(x, emb_table):
    b, seq = x.shape
    assert (b, seq) == (_BATCH, _SEQ) and emb_table.shape == (_VOCAB, _SIZE)
    x2d = x.astype(jnp.int32).reshape(b * seq // _GATHER, _GATHER)
    pe = jnp.asarray(_PE)

    run = pl.kernel(
        _body,
        out_type=jax.ShapeDtypeStruct((b * seq, _SIZE), jnp.float32),
        mesh=plsc.VectorSubcoreMesh(core_axis_name="c", subcore_axis_name="s"),
        compiler_params=pltpu.CompilerParams(
            use_tc_tiling_on_sc=False, skip_device_barrier=True),
        scratch_types=[
            pltpu.VMEM((_IDX_ROWS, _GATHER), jnp.int32),
            pltpu.VMEM((_ROWS_PER_CHUNK, _SIZE), jnp.float32),
            pltpu.VMEM((_ROWS_PER_CHUNK, _SIZE), jnp.float32),
            pltpu.VMEM((_SEQ, _SIZE), jnp.float32),
            pltpu.SemaphoreType.DMA,
            pltpu.SemaphoreType.DMA,
            pltpu.SemaphoreType.DMA,
        ],
    )
    out = run(emb_table, x2d, pe)
    return out.reshape(b, seq, _SIZE)
